# baseline (speedup 1.0000x reference)
import functools

import jax
import jax.numpy as jnp
from jax import lax
from jax.experimental import pallas as pl
from jax.experimental.pallas import tpu as pltpu
from jax.experimental.pallas import tpu_sc as plsc


def _dist_body(pos_c_ref, pos_r_ref, dist_ref):
    B, P, _ = pos_c_ref.shape
    for b in range(B):
        px_c = pos_c_ref[b, :, 0:1]
        py_c = pos_c_ref[b, :, 1:2]
        px_r = pos_r_ref[b, 0:1, :]
        py_r = pos_r_ref[b, 1:2, :]
        dx = px_c - px_r
        dy = py_c - py_r
        dist_ref[b * P:(b + 1) * P, :] = jnp.sqrt(dx * dx + dy * dy)


def _sel_body(dist_hbm, sel_hbm, dist_v, sel_v, *, nc, gpw, P, K):
    c = lax.axis_index("c")
    s = lax.axis_index("s")
    wid = s * nc + c
    base = wid * gpw * P * P
    pltpu.sync_copy(dist_hbm.at[pl.ds(base, gpw * P * P)], dist_v)

    def pair_body(pi, carry):
        for half in range(2):
            goff = (2 * pi + half) * P * P

            def chunk_body(ci, carry2):
                off = goff + ci * 16
                djs = [dist_v[pl.ds(off + j * P, 16)] for j in range(K)]
                accs = [jnp.zeros((16,), jnp.float32) for _ in range(K)]
                for k in range(P):
                    dk = (djs[k] if k < K
                          else dist_v[pl.ds(off + k * P, 16)])
                    for j in range(K):
                        if k == j:
                            continue
                        m = (dk <= djs[j]) if k < j else (dk < djs[j])
                        accs[j] = accs[j] + jnp.where(m, 1.0, 0.0)
                for j in range(K):
                    sel_v[j, pl.ds(half * P + ci * 16, 16)] = accs[j]
                return carry2

            lax.fori_loop(0, P // 16, chunk_body, 0)
        pltpu.sync_copy(sel_v,
                        sel_hbm.at[:, pl.ds((wid * gpw + 2 * pi) * P, 2 * P)])
        return carry

    lax.fori_loop(0, gpw // 2, pair_body, 0)


def _sel_ranks(dist_flat, P, K):
    N = dist_flat.shape[0] // P
    G = N // P
    info = plsc.get_sparse_core_info()
    nc, ns = info.num_cores, info.num_subcores
    gpw = G // (nc * ns)
    mesh = plsc.VectorSubcoreMesh(core_axis_name="c", subcore_axis_name="s")
    fn = functools.partial(
        pl.kernel,
        mesh=mesh,
        out_type=jax.ShapeDtypeStruct((K, N), jnp.float32),
        scratch_types=[
            pltpu.VMEM((gpw * P * P,), jnp.float32),
            pltpu.VMEM((K, 2 * P), jnp.float32),
        ],
    )(functools.partial(_sel_body, nc=nc, gpw=gpw, P=P, K=K))
    return fn(dist_flat)


def _stage_a(sel_ref, h_ref, w1_ref, b1_ref, y1_ref, s1_ref, q1_ref, x_s):
    B, P, H = h_ref.shape
    BP = B * P
    K = sel_ref.shape[0]
    hh = h_ref[...].reshape(BP, H).astype(jnp.bfloat16)
    qcol = lax.broadcasted_iota(jnp.int32, (BP, 1), 0).astype(jnp.float32)
    boff = ((lax.broadcasted_iota(jnp.int32, (1, BP), 1) // P) * P
            ).astype(jnp.float32)

    for j in range(K):
        tgt = sel_ref[j:j + 1, :] + boff
        ohT = (qcol == tgt).astype(jnp.bfloat16)
        gj = lax.dot_general(ohT, hh, (((0,), (0,)), ((), ())),
                             preferred_element_type=jnp.float32)
        x_s[:, j * H:(j + 1) * H] = gj.astype(jnp.bfloat16)

    y = jnp.dot(x_s[...], w1_ref[...],
                preferred_element_type=jnp.float32) + b1_ref[...]
    y1_ref[...] = y.reshape(B, P, y.shape[-1]).astype(jnp.bfloat16)

    @pl.when(pl.program_id(0) == 0)
    def _():
        s1_ref[...] = jnp.zeros_like(s1_ref)
        q1_ref[...] = jnp.zeros_like(q1_ref)

    s1_ref[...] += jnp.sum(y, axis=0, keepdims=True)
    q1_ref[...] += jnp.sum(y * y, axis=0, keepdims=True)


def _stage_b(y1_ref, s1_ref, q1_ref, g1_ref, be1_ref, w2_ref, b2_ref,
             y2_ref, s2_ref, q2_ref, *, n):
    m = s1_ref[...] / n
    v = q1_ref[...] / n - m * m
    scale = g1_ref[...] * lax.rsqrt(v + 1e-5)
    a = (y1_ref[...].astype(jnp.float32) - m) * scale + be1_ref[...]
    a = jnp.where(a >= 0, a, 0.01 * a).astype(jnp.bfloat16)
    y = jnp.dot(a, w2_ref[...], preferred_element_type=jnp.float32) + b2_ref[...]
    y2_ref[...] = y.astype(jnp.bfloat16)

    @pl.when(pl.program_id(0) == 0)
    def _():
        s2_ref[...] = jnp.zeros_like(s2_ref)
        q2_ref[...] = jnp.zeros_like(q2_ref)

    s2_ref[...] += jnp.sum(y, axis=0, keepdims=True)
    q2_ref[...] += jnp.sum(y * y, axis=0, keepdims=True)


def _stage_c(y2_ref, s2_ref, q2_ref, g2_ref, be2_ref, out_ref, *, n):
    m = s2_ref[...] / n
    v = q2_ref[...] / n - m * m
    scale = g2_ref[...] * lax.rsqrt(v + 1e-5)
    a = (y2_ref[...].astype(jnp.float32) - m) * scale + be2_ref[...]
    out_ref[...] = jnp.where(a >= 0, a, 0.01 * a)


def kernel(h_states, seq_start_end, last_pos, W1, b1, g1, be1, W2, b2, g2, be2):
    G = seq_start_end.shape[0]
    N, H = h_states.shape
    P = N // G
    D1 = W1.shape[1]
    D2 = W2.shape[1]
    K = W1.shape[0] // H
    B = 4 if G % 4 == 0 else 1
    NB = G // B

    pos3 = last_pos.reshape(G, P, 2)
    pos3t = pos3.transpose(0, 2, 1)
    BD = 8 if G % 8 == 0 else 1
    dist = pl.pallas_call(
        _dist_body,
        grid=(G // BD,),
        in_specs=[
            pl.BlockSpec((BD, P, 2), lambda g: (g, 0, 0)),
            pl.BlockSpec((BD, 2, P), lambda g: (g, 0, 0)),
        ],
        out_specs=pl.BlockSpec((BD * P, P), lambda g: (g, 0)),
        out_shape=jax.ShapeDtypeStruct((N, P), jnp.float32),
    )(pos3, pos3t)

    sel = _sel_ranks(dist.reshape(N * P), P, K)

    h3 = h_states.reshape(G, P, H)

    y1, s1, q1 = pl.pallas_call(
        _stage_a,
        grid=(NB,),
        in_specs=[
            pl.BlockSpec((K, B * P), lambda g: (0, g)),
            pl.BlockSpec((B, P, H), lambda g: (g, 0, 0)),
            pl.BlockSpec((K * H, D1), lambda g: (0, 0)),
            pl.BlockSpec((1, D1), lambda g: (0, 0)),
        ],
        out_specs=[
            pl.BlockSpec((B, P, D1), lambda g: (g, 0, 0)),
            pl.BlockSpec((1, D1), lambda g: (0, 0)),
            pl.BlockSpec((1, D1), lambda g: (0, 0)),
        ],
        out_shape=[
            jax.ShapeDtypeStruct((G, P, D1), jnp.bfloat16),
            jax.ShapeDtypeStruct((1, D1), jnp.float32),
            jax.ShapeDtypeStruct((1, D1), jnp.float32),
        ],
        scratch_shapes=[
            pltpu.VMEM((B * P, K * H), jnp.bfloat16),
        ],
    )(sel, h3, W1.astype(jnp.bfloat16), b1.reshape(1, D1))

    y1f = y1.reshape(N, D1)
    RB = 512

    y2, s2, q2 = pl.pallas_call(
        functools.partial(_stage_b, n=float(N)),
        grid=(N // RB,),
        in_specs=[
            pl.BlockSpec((RB, D1), lambda i: (i, 0)),
            pl.BlockSpec((1, D1), lambda i: (0, 0)),
            pl.BlockSpec((1, D1), lambda i: (0, 0)),
            pl.BlockSpec((1, D1), lambda i: (0, 0)),
            pl.BlockSpec((1, D1), lambda i: (0, 0)),
            pl.BlockSpec((D1, D2), lambda i: (0, 0)),
            pl.BlockSpec((1, D2), lambda i: (0, 0)),
        ],
        out_specs=[
            pl.BlockSpec((RB, D2), lambda i: (i, 0)),
            pl.BlockSpec((1, D2), lambda i: (0, 0)),
            pl.BlockSpec((1, D2), lambda i: (0, 0)),
        ],
        out_shape=[
            jax.ShapeDtypeStruct((N, D2), jnp.bfloat16),
            jax.ShapeDtypeStruct((1, D2), jnp.float32),
            jax.ShapeDtypeStruct((1, D2), jnp.float32),
        ],
    )(y1f, s1, q1, g1.reshape(1, D1), be1.reshape(1, D1),
      W2.astype(jnp.bfloat16), b2.reshape(1, D2))

    out = pl.pallas_call(
        functools.partial(_stage_c, n=float(N)),
        grid=(N // RB,),
        in_specs=[
            pl.BlockSpec((RB, D2), lambda i: (i, 0)),
            pl.BlockSpec((1, D2), lambda i: (0, 0)),
            pl.BlockSpec((1, D2), lambda i: (0, 0)),
            pl.BlockSpec((1, D2), lambda i: (0, 0)),
            pl.BlockSpec((1, D2), lambda i: (0, 0)),
        ],
        out_specs=pl.BlockSpec((RB, D2), lambda i: (i, 0)),
        out_shape=jax.ShapeDtypeStruct((N, D2), jnp.float32),
    )(y2, s2, q2, g2.reshape(1, D2), be2.reshape(1, D2))

    return out


# speedup vs baseline: 1.0722x; 1.0722x over previous
import functools

import jax
import jax.numpy as jnp
from jax import lax
from jax.experimental import pallas as pl
from jax.experimental.pallas import tpu as pltpu
from jax.experimental.pallas import tpu_sc as plsc


def _dist_body(pos_c_ref, pos_r_ref, dist_ref):
    B, P, _ = pos_c_ref.shape
    for b in range(B):
        px_c = pos_c_ref[b, :, 0:1]
        py_c = pos_c_ref[b, :, 1:2]
        px_r = pos_r_ref[b, 0:1, :]
        py_r = pos_r_ref[b, 1:2, :]
        dx = px_c - px_r
        dy = py_c - py_r
        dist_ref[b * P:(b + 1) * P, :] = jnp.sqrt(dx * dx + dy * dy)


def _sel_body(dist_hbm, sel_hbm, dist_v, sel_v, *, nc, gpw, P, K):
    c = lax.axis_index("c")
    s = lax.axis_index("s")
    wid = s * nc + c
    base = wid * gpw * P * P
    pltpu.sync_copy(dist_hbm.at[pl.ds(base, gpw * P * P)], dist_v)

    def pair_body(pi, carry):
        for half in range(2):
            goff = (2 * pi + half) * P * P

            def chunk_body(ci, carry2):
                off = goff + ci * 16
                djs = [dist_v[pl.ds(off + j * P, 16)] for j in range(K)]
                accs = [jnp.zeros((16,), jnp.float32) for _ in range(K)]
                for k in range(P):
                    dk = (djs[k] if k < K
                          else dist_v[pl.ds(off + k * P, 16)])
                    for j in range(K):
                        if k == j:
                            continue
                        m = (dk <= djs[j]) if k < j else (dk < djs[j])
                        accs[j] = accs[j] + jnp.where(m, 1.0, 0.0)
                for j in range(K):
                    sel_v[j, pl.ds(half * P + ci * 16, 16)] = accs[j]
                return carry2

            lax.fori_loop(0, P // 16, chunk_body, 0)
        pltpu.sync_copy(sel_v,
                        sel_hbm.at[:, pl.ds((wid * gpw + 2 * pi) * P, 2 * P)])
        return carry

    lax.fori_loop(0, gpw // 2, pair_body, 0)


def _sel_ranks(dist_flat, P, K):
    N = dist_flat.shape[0] // P
    G = N // P
    info = plsc.get_sparse_core_info()
    nc, ns = info.num_cores, info.num_subcores
    gpw = G // (nc * ns)
    mesh = plsc.VectorSubcoreMesh(core_axis_name="c", subcore_axis_name="s")
    fn = functools.partial(
        pl.kernel,
        mesh=mesh,
        out_type=jax.ShapeDtypeStruct((K, N), jnp.float32),
        scratch_types=[
            pltpu.VMEM((gpw * P * P,), jnp.float32),
            pltpu.VMEM((K, 2 * P), jnp.float32),
        ],
    )(functools.partial(_sel_body, nc=nc, gpw=gpw, P=P, K=K))
    return fn(dist_flat)


def _stage_a(sel_ref, h_ref, w1_ref, b1_ref, y1_ref, s1_ref, q1_ref, x_s):
    B, P, H = h_ref.shape
    BP = B * P
    K = sel_ref.shape[0]
    hh = h_ref[...].reshape(BP, H).astype(jnp.bfloat16)
    qcol = lax.broadcasted_iota(jnp.int32, (BP, 1), 0).astype(jnp.float32)
    boff = ((lax.broadcasted_iota(jnp.int32, (1, BP), 1) // P) * P
            ).astype(jnp.float32)

    for j in range(K):
        tgt = sel_ref[j:j + 1, :] + boff
        ohT = (qcol == tgt).astype(jnp.bfloat16)
        gj = lax.dot_general(ohT, hh, (((0,), (0,)), ((), ())),
                             preferred_element_type=jnp.float32)
        x_s[:, j * H:(j + 1) * H] = gj.astype(jnp.bfloat16)

    y = jnp.dot(x_s[...], w1_ref[...],
                preferred_element_type=jnp.float32) + b1_ref[...]
    y1_ref[...] = y.reshape(B, P, y.shape[-1]).astype(jnp.bfloat16)

    @pl.when(pl.program_id(0) == 0)
    def _():
        s1_ref[...] = jnp.zeros_like(s1_ref)
        q1_ref[...] = jnp.zeros_like(q1_ref)

    s1_ref[...] += jnp.sum(y, axis=0, keepdims=True)
    q1_ref[...] += jnp.sum(y * y, axis=0, keepdims=True)


def _stage_b(y1_ref, s1_ref, q1_ref, g1_ref, be1_ref, w2_ref, b2_ref,
             y2_ref, s2_ref, q2_ref, *, n):
    m = s1_ref[...] / n
    v = q1_ref[...] / n - m * m
    scale = g1_ref[...] * lax.rsqrt(v + 1e-5)
    a = (y1_ref[...].astype(jnp.float32) - m) * scale + be1_ref[...]
    a = jnp.where(a >= 0, a, 0.01 * a)
    y = jnp.dot(a, w2_ref[...], preferred_element_type=jnp.float32) + b2_ref[...]
    y2_ref[...] = y.astype(jnp.bfloat16)

    @pl.when(pl.program_id(0) == 0)
    def _():
        s2_ref[...] = jnp.zeros_like(s2_ref)
        q2_ref[...] = jnp.zeros_like(q2_ref)

    s2_ref[...] += jnp.sum(y, axis=0, keepdims=True)
    q2_ref[...] += jnp.sum(y * y, axis=0, keepdims=True)


def _stage_c(y2_ref, s2_ref, q2_ref, g2_ref, be2_ref, out_ref, *, n):
    m = s2_ref[...] / n
    v = q2_ref[...] / n - m * m
    scale = g2_ref[...] * lax.rsqrt(v + 1e-5)
    a = (y2_ref[...].astype(jnp.float32) - m) * scale + be2_ref[...]
    out_ref[...] = jnp.where(a >= 0, a, 0.01 * a)


def kernel(h_states, seq_start_end, last_pos, W1, b1, g1, be1, W2, b2, g2, be2):
    G = seq_start_end.shape[0]
    N, H = h_states.shape
    P = N // G
    D1 = W1.shape[1]
    D2 = W2.shape[1]
    K = W1.shape[0] // H
    B = 4 if G % 4 == 0 else 1
    NB = G // B

    pos3 = last_pos.reshape(G, P, 2)
    pos3t = pos3.transpose(0, 2, 1)
    BD = 8 if G % 8 == 0 else 1
    dist = pl.pallas_call(
        _dist_body,
        grid=(G // BD,),
        in_specs=[
            pl.BlockSpec((BD, P, 2), lambda g: (g, 0, 0)),
            pl.BlockSpec((BD, 2, P), lambda g: (g, 0, 0)),
        ],
        out_specs=pl.BlockSpec((BD * P, P), lambda g: (g, 0)),
        out_shape=jax.ShapeDtypeStruct((N, P), jnp.float32),
    )(pos3, pos3t)

    sel = _sel_ranks(dist.reshape(N * P), P, K)

    h3 = h_states.reshape(G, P, H)

    y1, s1, q1 = pl.pallas_call(
        _stage_a,
        grid=(NB,),
        in_specs=[
            pl.BlockSpec((K, B * P), lambda g: (0, g)),
            pl.BlockSpec((B, P, H), lambda g: (g, 0, 0)),
            pl.BlockSpec((K * H, D1), lambda g: (0, 0)),
            pl.BlockSpec((1, D1), lambda g: (0, 0)),
        ],
        out_specs=[
            pl.BlockSpec((B, P, D1), lambda g: (g, 0, 0)),
            pl.BlockSpec((1, D1), lambda g: (0, 0)),
            pl.BlockSpec((1, D1), lambda g: (0, 0)),
        ],
        out_shape=[
            jax.ShapeDtypeStruct((G, P, D1), jnp.bfloat16),
            jax.ShapeDtypeStruct((1, D1), jnp.float32),
            jax.ShapeDtypeStruct((1, D1), jnp.float32),
        ],
        scratch_shapes=[
            pltpu.VMEM((B * P, K * H), jnp.bfloat16),
        ],
    )(sel, h3, W1.astype(jnp.bfloat16), b1.reshape(1, D1))

    y1f = y1.reshape(N, D1)
    RB = 1024

    y2, s2, q2 = pl.pallas_call(
        functools.partial(_stage_b, n=float(N)),
        grid=(N // RB,),
        in_specs=[
            pl.BlockSpec((RB, D1), lambda i: (i, 0)),
            pl.BlockSpec((1, D1), lambda i: (0, 0)),
            pl.BlockSpec((1, D1), lambda i: (0, 0)),
            pl.BlockSpec((1, D1), lambda i: (0, 0)),
            pl.BlockSpec((1, D1), lambda i: (0, 0)),
            pl.BlockSpec((D1, D2), lambda i: (0, 0)),
            pl.BlockSpec((1, D2), lambda i: (0, 0)),
        ],
        out_specs=[
            pl.BlockSpec((RB, D2), lambda i: (i, 0)),
            pl.BlockSpec((1, D2), lambda i: (0, 0)),
            pl.BlockSpec((1, D2), lambda i: (0, 0)),
        ],
        out_shape=[
            jax.ShapeDtypeStruct((N, D2), jnp.bfloat16),
            jax.ShapeDtypeStruct((1, D2), jnp.float32),
            jax.ShapeDtypeStruct((1, D2), jnp.float32),
        ],
    )(y1f, s1, q1, g1.reshape(1, D1), be1.reshape(1, D1),
      W2, b2.reshape(1, D2))

    out = pl.pallas_call(
        functools.partial(_stage_c, n=float(N)),
        grid=(N // RB,),
        in_specs=[
            pl.BlockSpec((RB, D2), lambda i: (i, 0)),
            pl.BlockSpec((1, D2), lambda i: (0, 0)),
            pl.BlockSpec((1, D2), lambda i: (0, 0)),
            pl.BlockSpec((1, D2), lambda i: (0, 0)),
            pl.BlockSpec((1, D2), lambda i: (0, 0)),
        ],
        out_specs=pl.BlockSpec((RB, D2), lambda i: (i, 0)),
        out_shape=jax.ShapeDtypeStruct((N, D2), jnp.float32),
    )(y2, s2, q2, g2.reshape(1, D2), be2.reshape(1, D2))

    return out


# speedup vs baseline: 1.1216x; 1.0461x over previous
import functools

import jax
import jax.numpy as jnp
from jax import lax
from jax.experimental import pallas as pl
from jax.experimental.pallas import tpu as pltpu
from jax.experimental.pallas import tpu_sc as plsc


def _dist_body(pos_c_ref, pos_r_ref, dist_ref):
    B, P, _ = pos_c_ref.shape
    for b in range(B):
        px_c = pos_c_ref[b, :, 0:1]
        py_c = pos_c_ref[b, :, 1:2]
        px_r = pos_r_ref[b, 0:1, :]
        py_r = pos_r_ref[b, 1:2, :]
        dx = px_c - px_r
        dy = py_c - py_r
        dist_ref[b * P:(b + 1) * P, :] = jnp.sqrt(dx * dx + dy * dy)


def _sel_body(dist_hbm, sel_hbm, dist_v, sel_v, *, nc, gpw, P, K):
    c = lax.axis_index("c")
    s = lax.axis_index("s")
    wid = s * nc + c
    base = wid * gpw * P * P
    pltpu.sync_copy(dist_hbm.at[pl.ds(base, gpw * P * P)], dist_v)

    def pair_body(pi, carry):
        for half in range(2):
            goff = (2 * pi + half) * P * P

            def chunk_body(ci, carry2):
                off = goff + ci * 16
                djs = [dist_v[pl.ds(off + j * P, 16)] for j in range(K)]
                accs = [jnp.zeros((16,), jnp.float32) for _ in range(K)]
                for k in range(P):
                    dk = (djs[k] if k < K
                          else dist_v[pl.ds(off + k * P, 16)])
                    for j in range(K):
                        if k == j:
                            continue
                        m = (dk <= djs[j]) if k < j else (dk < djs[j])
                        accs[j] = accs[j] + jnp.where(m, 1.0, 0.0)
                for j in range(K):
                    sel_v[j, pl.ds(half * P + ci * 16, 16)] = accs[j]
                return carry2

            lax.fori_loop(0, P // 16, chunk_body, 0)
        pltpu.sync_copy(sel_v,
                        sel_hbm.at[:, pl.ds((wid * gpw + 2 * pi) * P, 2 * P)])
        return carry

    lax.fori_loop(0, gpw // 2, pair_body, 0)


def _sel_ranks(dist_flat, P, K):
    N = dist_flat.shape[0] // P
    G = N // P
    info = plsc.get_sparse_core_info()
    nc, ns = info.num_cores, info.num_subcores
    gpw = G // (nc * ns)
    mesh = plsc.VectorSubcoreMesh(core_axis_name="c", subcore_axis_name="s")
    fn = functools.partial(
        pl.kernel,
        mesh=mesh,
        out_type=jax.ShapeDtypeStruct((K, N), jnp.float32),
        scratch_types=[
            pltpu.VMEM((gpw * P * P,), jnp.float32),
            pltpu.VMEM((K, 2 * P), jnp.float32),
        ],
    )(functools.partial(_sel_body, nc=nc, gpw=gpw, P=P, K=K))
    return fn(dist_flat)


def _stage_a(sel_ref, h_ref, w1_ref, b1_ref, y1_ref, s1_ref, q1_ref, x_s):
    B, P, H = h_ref.shape
    BP = B * P
    K = sel_ref.shape[0]
    hh = h_ref[...].reshape(BP, H).astype(jnp.bfloat16)
    qcol = lax.broadcasted_iota(jnp.int32, (BP, 1), 0).astype(jnp.float32)
    boff = ((lax.broadcasted_iota(jnp.int32, (1, BP), 1) // P) * P
            ).astype(jnp.float32)

    for j in range(K):
        tgt = sel_ref[j:j + 1, :] + boff
        ohT = (qcol == tgt).astype(jnp.bfloat16)
        gj = lax.dot_general(ohT, hh, (((0,), (0,)), ((), ())),
                             preferred_element_type=jnp.float32)
        x_s[:, j * H:(j + 1) * H] = gj.astype(jnp.bfloat16)

    y = jnp.dot(x_s[...], w1_ref[...],
                preferred_element_type=jnp.float32) + b1_ref[...]
    y1_ref[...] = y.reshape(B, P, y.shape[-1]).astype(jnp.bfloat16)

    @pl.when(pl.program_id(0) == 0)
    def _():
        s1_ref[...] = jnp.zeros_like(s1_ref)
        q1_ref[...] = jnp.zeros_like(q1_ref)

    s1_ref[...] += jnp.sum(y, axis=0, keepdims=True)
    q1_ref[...] += jnp.sum(y * y, axis=0, keepdims=True)


def _stage_b(y1_ref, s1_ref, q1_ref, g1_ref, be1_ref, w2_ref, b2_ref,
             y2_ref, s2_ref, q2_ref, *, n):
    m = s1_ref[...] / n
    v = q1_ref[...] / n - m * m
    scale = g1_ref[...] * lax.rsqrt(v + 1e-5)
    a = (y1_ref[...].astype(jnp.float32) - m) * scale + be1_ref[...]
    a = jnp.where(a >= 0, a, 0.01 * a)
    y = jnp.dot(a, w2_ref[...], preferred_element_type=jnp.float32) + b2_ref[...]
    y2_ref[...] = y.astype(jnp.bfloat16)

    @pl.when(pl.program_id(0) == 0)
    def _():
        s2_ref[...] = jnp.zeros_like(s2_ref)
        q2_ref[...] = jnp.zeros_like(q2_ref)

    s2_ref[...] += jnp.sum(y, axis=0, keepdims=True)
    q2_ref[...] += jnp.sum(y * y, axis=0, keepdims=True)


def _stage_c(y2_ref, s2_ref, q2_ref, g2_ref, be2_ref, out_ref, *, n):
    m = s2_ref[...] / n
    v = q2_ref[...] / n - m * m
    scale = g2_ref[...] * lax.rsqrt(v + 1e-5)
    a = (y2_ref[...].astype(jnp.float32) - m) * scale + be2_ref[...]
    out_ref[...] = jnp.where(a >= 0, a, 0.01 * a)


def kernel(h_states, seq_start_end, last_pos, W1, b1, g1, be1, W2, b2, g2, be2):
    G = seq_start_end.shape[0]
    N, H = h_states.shape
    P = N // G
    D1 = W1.shape[1]
    D2 = W2.shape[1]
    K = W1.shape[0] // H
    B = 4 if G % 4 == 0 else 1
    NB = G // B

    pos3 = last_pos.reshape(G, P, 2)
    pos3t = pos3.transpose(0, 2, 1)
    BD = 8 if G % 8 == 0 else 1
    dist = pl.pallas_call(
        _dist_body,
        grid=(G // BD,),
        in_specs=[
            pl.BlockSpec((BD, P, 2), lambda g: (g, 0, 0)),
            pl.BlockSpec((BD, 2, P), lambda g: (g, 0, 0)),
        ],
        out_specs=pl.BlockSpec((BD * P, P), lambda g: (g, 0)),
        out_shape=jax.ShapeDtypeStruct((N, P), jnp.float32),
    )(pos3, pos3t)

    sel = _sel_ranks(dist.reshape(N * P), P, K)

    h3 = h_states.reshape(G, P, H)

    y1, s1, q1 = pl.pallas_call(
        _stage_a,
        grid=(NB,),
        in_specs=[
            pl.BlockSpec((K, B * P), lambda g: (0, g)),
            pl.BlockSpec((B, P, H), lambda g: (g, 0, 0)),
            pl.BlockSpec((K * H, D1), lambda g: (0, 0)),
            pl.BlockSpec((1, D1), lambda g: (0, 0)),
        ],
        out_specs=[
            pl.BlockSpec((B, P, D1), lambda g: (g, 0, 0)),
            pl.BlockSpec((1, D1), lambda g: (0, 0)),
            pl.BlockSpec((1, D1), lambda g: (0, 0)),
        ],
        out_shape=[
            jax.ShapeDtypeStruct((G, P, D1), jnp.bfloat16),
            jax.ShapeDtypeStruct((1, D1), jnp.float32),
            jax.ShapeDtypeStruct((1, D1), jnp.float32),
        ],
        scratch_shapes=[
            pltpu.VMEM((B * P, K * H), jnp.bfloat16),
        ],
    )(sel, h3, W1.astype(jnp.bfloat16), b1.reshape(1, D1))

    y1f = y1.reshape(N, D1)
    RB = 2048

    y2, s2, q2 = pl.pallas_call(
        functools.partial(_stage_b, n=float(N)),
        grid=(N // RB,),
        in_specs=[
            pl.BlockSpec((RB, D1), lambda i: (i, 0)),
            pl.BlockSpec((1, D1), lambda i: (0, 0)),
            pl.BlockSpec((1, D1), lambda i: (0, 0)),
            pl.BlockSpec((1, D1), lambda i: (0, 0)),
            pl.BlockSpec((1, D1), lambda i: (0, 0)),
            pl.BlockSpec((D1, D2), lambda i: (0, 0)),
            pl.BlockSpec((1, D2), lambda i: (0, 0)),
        ],
        out_specs=[
            pl.BlockSpec((RB, D2), lambda i: (i, 0)),
            pl.BlockSpec((1, D2), lambda i: (0, 0)),
            pl.BlockSpec((1, D2), lambda i: (0, 0)),
        ],
        out_shape=[
            jax.ShapeDtypeStruct((N, D2), jnp.bfloat16),
            jax.ShapeDtypeStruct((1, D2), jnp.float32),
            jax.ShapeDtypeStruct((1, D2), jnp.float32),
        ],
    )(y1f, s1, q1, g1.reshape(1, D1), be1.reshape(1, D1),
      W2, b2.reshape(1, D2))

    out = pl.pallas_call(
        functools.partial(_stage_c, n=float(N)),
        grid=(N // RB,),
        in_specs=[
            pl.BlockSpec((RB, D2), lambda i: (i, 0)),
            pl.BlockSpec((1, D2), lambda i: (0, 0)),
            pl.BlockSpec((1, D2), lambda i: (0, 0)),
            pl.BlockSpec((1, D2), lambda i: (0, 0)),
            pl.BlockSpec((1, D2), lambda i: (0, 0)),
        ],
        out_specs=pl.BlockSpec((RB, D2), lambda i: (i, 0)),
        out_shape=jax.ShapeDtypeStruct((N, D2), jnp.float32),
    )(y2, s2, q2, g2.reshape(1, D2), be2.reshape(1, D2))

    return out


# speedup vs baseline: 1.1276x; 1.0054x over previous
import functools

import jax
import jax.numpy as jnp
from jax import lax
from jax.experimental import pallas as pl
from jax.experimental.pallas import tpu as pltpu
from jax.experimental.pallas import tpu_sc as plsc


def _dist_body(pos_c_ref, pos_r_ref, dist_ref):
    B, P, _ = pos_c_ref.shape
    for b in range(B):
        px_c = pos_c_ref[b, :, 0:1]
        py_c = pos_c_ref[b, :, 1:2]
        px_r = pos_r_ref[b, 0:1, :]
        py_r = pos_r_ref[b, 1:2, :]
        dx = px_c - px_r
        dy = py_c - py_r
        dist_ref[b * P:(b + 1) * P, :] = jnp.sqrt(dx * dx + dy * dy)


def _sel_body(dist_hbm, sel_hbm, dist_v, sel_v, *, nc, gpw, P, K):
    c = lax.axis_index("c")
    s = lax.axis_index("s")
    wid = s * nc + c
    base = wid * gpw * P * P
    pltpu.sync_copy(dist_hbm.at[pl.ds(base, gpw * P * P)], dist_v)

    def pair_body(pi, carry):
        for half in range(2):
            goff = (2 * pi + half) * P * P

            def chunk_body(ci, carry2):
                off = goff + ci * 16
                djs = [dist_v[pl.ds(off + j * P, 16)] for j in range(K)]
                accs = [jnp.zeros((16,), jnp.float32) for _ in range(K)]
                for k in range(P):
                    dk = (djs[k] if k < K
                          else dist_v[pl.ds(off + k * P, 16)])
                    for j in range(K):
                        if k == j:
                            continue
                        m = (dk <= djs[j]) if k < j else (dk < djs[j])
                        accs[j] = accs[j] + jnp.where(m, 1.0, 0.0)
                for j in range(K):
                    sel_v[j, pl.ds(half * P + ci * 16, 16)] = accs[j]
                return carry2

            lax.fori_loop(0, P // 16, chunk_body, 0)
        pltpu.sync_copy(sel_v,
                        sel_hbm.at[:, pl.ds((wid * gpw + 2 * pi) * P, 2 * P)])
        return carry

    lax.fori_loop(0, gpw // 2, pair_body, 0)


def _sel_ranks(dist_flat, P, K):
    N = dist_flat.shape[0] // P
    G = N // P
    info = plsc.get_sparse_core_info()
    nc, ns = info.num_cores, info.num_subcores
    gpw = G // (nc * ns)
    mesh = plsc.VectorSubcoreMesh(core_axis_name="c", subcore_axis_name="s")
    fn = functools.partial(
        pl.kernel,
        mesh=mesh,
        out_type=jax.ShapeDtypeStruct((K, N), jnp.float32),
        scratch_types=[
            pltpu.VMEM((gpw * P * P,), jnp.float32),
            pltpu.VMEM((K, 2 * P), jnp.float32),
        ],
    )(functools.partial(_sel_body, nc=nc, gpw=gpw, P=P, K=K))
    return fn(dist_flat)


def _stage_a(sel_ref, h_ref, w1_ref, b1_ref, y1_ref, s1_ref, q1_ref, x_s):
    B, P, H = h_ref.shape
    BP = B * P
    K = sel_ref.shape[0]
    hh = h_ref[...].reshape(BP, H).astype(jnp.bfloat16)
    qcol = lax.broadcasted_iota(jnp.int32, (BP, 1), 0).astype(jnp.float32)
    boff = ((lax.broadcasted_iota(jnp.int32, (1, BP), 1) // P) * P
            ).astype(jnp.float32)

    for j in range(K):
        tgt = sel_ref[j:j + 1, :] + boff
        ohT = (qcol == tgt).astype(jnp.bfloat16)
        gj = lax.dot_general(ohT, hh, (((0,), (0,)), ((), ())),
                             preferred_element_type=jnp.float32)
        x_s[:, j * H:(j + 1) * H] = gj.astype(jnp.bfloat16)

    y = jnp.dot(x_s[...], w1_ref[...],
                preferred_element_type=jnp.float32) + b1_ref[...]
    y1_ref[...] = y.reshape(B, P, y.shape[-1]).astype(jnp.bfloat16)

    @pl.when(pl.program_id(0) == 0)
    def _():
        s1_ref[...] = jnp.zeros_like(s1_ref)
        q1_ref[...] = jnp.zeros_like(q1_ref)

    s1_ref[...] += jnp.sum(y, axis=0, keepdims=True)
    q1_ref[...] += jnp.sum(y * y, axis=0, keepdims=True)


def _stage_b(y1_ref, s1_ref, q1_ref, g1_ref, be1_ref, w2_ref, b2_ref,
             y2_ref, s2_ref, q2_ref, *, n):
    m = s1_ref[...] / n
    v = q1_ref[...] / n - m * m
    scale = g1_ref[...] * lax.rsqrt(v + 1e-5)
    a = (y1_ref[...].astype(jnp.float32) - m) * scale + be1_ref[...]
    a = jnp.where(a >= 0, a, 0.01 * a)
    y = jnp.dot(a, w2_ref[...], preferred_element_type=jnp.float32) + b2_ref[...]
    y2_ref[...] = y.astype(jnp.bfloat16)

    @pl.when(pl.program_id(0) == 0)
    def _():
        s2_ref[...] = jnp.zeros_like(s2_ref)
        q2_ref[...] = jnp.zeros_like(q2_ref)

    s2_ref[...] += jnp.sum(y, axis=0, keepdims=True)
    q2_ref[...] += jnp.sum(y * y, axis=0, keepdims=True)


def _stage_c(y2_ref, s2_ref, q2_ref, g2_ref, be2_ref, out_ref, *, n):
    m = s2_ref[...] / n
    v = q2_ref[...] / n - m * m
    scale = g2_ref[...] * lax.rsqrt(v + 1e-5)
    a = (y2_ref[...].astype(jnp.float32) - m) * scale + be2_ref[...]
    out_ref[...] = jnp.where(a >= 0, a, 0.01 * a)


def kernel(h_states, seq_start_end, last_pos, W1, b1, g1, be1, W2, b2, g2, be2):
    G = seq_start_end.shape[0]
    N, H = h_states.shape
    P = N // G
    D1 = W1.shape[1]
    D2 = W2.shape[1]
    K = W1.shape[0] // H
    B = 4 if G % 4 == 0 else 1
    NB = G // B

    pos3 = last_pos.reshape(G, P, 2)
    pos3t = pos3.transpose(0, 2, 1)
    BD = 8 if G % 8 == 0 else 1
    dist = pl.pallas_call(
        _dist_body,
        grid=(G // BD,),
        in_specs=[
            pl.BlockSpec((BD, P, 2), lambda g: (g, 0, 0)),
            pl.BlockSpec((BD, 2, P), lambda g: (g, 0, 0)),
        ],
        out_specs=pl.BlockSpec((BD * P, P), lambda g: (g, 0)),
        out_shape=jax.ShapeDtypeStruct((N, P), jnp.float32),
    )(pos3, pos3t)

    sel = _sel_ranks(dist.reshape(N * P), P, K)

    h3 = h_states.reshape(G, P, H)

    y1, s1, q1 = pl.pallas_call(
        _stage_a,
        grid=(NB,),
        in_specs=[
            pl.BlockSpec((K, B * P), lambda g: (0, g)),
            pl.BlockSpec((B, P, H), lambda g: (g, 0, 0)),
            pl.BlockSpec((K * H, D1), lambda g: (0, 0)),
            pl.BlockSpec((1, D1), lambda g: (0, 0)),
        ],
        out_specs=[
            pl.BlockSpec((B, P, D1), lambda g: (g, 0, 0)),
            pl.BlockSpec((1, D1), lambda g: (0, 0)),
            pl.BlockSpec((1, D1), lambda g: (0, 0)),
        ],
        out_shape=[
            jax.ShapeDtypeStruct((G, P, D1), jnp.bfloat16),
            jax.ShapeDtypeStruct((1, D1), jnp.float32),
            jax.ShapeDtypeStruct((1, D1), jnp.float32),
        ],
        scratch_shapes=[
            pltpu.VMEM((B * P, K * H), jnp.bfloat16),
        ],
    )(sel, h3, W1.astype(jnp.bfloat16), b1.reshape(1, D1))

    y1f = y1.reshape(N, D1)
    RB = 4096

    y2, s2, q2 = pl.pallas_call(
        functools.partial(_stage_b, n=float(N)),
        grid=(N // RB,),
        in_specs=[
            pl.BlockSpec((RB, D1), lambda i: (i, 0)),
            pl.BlockSpec((1, D1), lambda i: (0, 0)),
            pl.BlockSpec((1, D1), lambda i: (0, 0)),
            pl.BlockSpec((1, D1), lambda i: (0, 0)),
            pl.BlockSpec((1, D1), lambda i: (0, 0)),
            pl.BlockSpec((D1, D2), lambda i: (0, 0)),
            pl.BlockSpec((1, D2), lambda i: (0, 0)),
        ],
        out_specs=[
            pl.BlockSpec((RB, D2), lambda i: (i, 0)),
            pl.BlockSpec((1, D2), lambda i: (0, 0)),
            pl.BlockSpec((1, D2), lambda i: (0, 0)),
        ],
        out_shape=[
            jax.ShapeDtypeStruct((N, D2), jnp.bfloat16),
            jax.ShapeDtypeStruct((1, D2), jnp.float32),
            jax.ShapeDtypeStruct((1, D2), jnp.float32),
        ],
    )(y1f, s1, q1, g1.reshape(1, D1), be1.reshape(1, D1),
      W2, b2.reshape(1, D2))

    out = pl.pallas_call(
        functools.partial(_stage_c, n=float(N)),
        grid=(N // RB,),
        in_specs=[
            pl.BlockSpec((RB, D2), lambda i: (i, 0)),
            pl.BlockSpec((1, D2), lambda i: (0, 0)),
            pl.BlockSpec((1, D2), lambda i: (0, 0)),
            pl.BlockSpec((1, D2), lambda i: (0, 0)),
            pl.BlockSpec((1, D2), lambda i: (0, 0)),
        ],
        out_specs=pl.BlockSpec((RB, D2), lambda i: (i, 0)),
        out_shape=jax.ShapeDtypeStruct((N, D2), jnp.float32),
    )(y2, s2, q2, g2.reshape(1, D2), be2.reshape(1, D2))

    return out
